# baseline (speedup 1.0000x reference)
import jax
import jax.numpy as jnp
from jax.experimental import pallas as pl

_GRID = 25
_U_BLK = 40960
_M_BLK = 4096


def _fused_kernel(x_ref, u_ref, w_ref, b_ref, uo_ref, mo_ref):
    uo_ref[...] = u_ref[...]
    acc = jax.lax.dot_general(
        w_ref[...], x_ref[...],
        dimension_numbers=(((0,), (1,)), ((), ())),
        preferred_element_type=jnp.float32,
    )
    mo_ref[...] = jnp.maximum(acc + b_ref[...], 0.0)


def kernel(movie_x, user_emb_weight, W, b):
    n, f = movie_x.shape
    nu, e = user_emb_weight.shape
    u_t = user_emb_weight.T
    user_t, movie_t = pl.pallas_call(
        _fused_kernel,
        grid=(_GRID,),
        in_specs=[
            pl.BlockSpec((_M_BLK, f), lambda i: (i, 0)),
            pl.BlockSpec((e, _U_BLK), lambda i: (0, i)),
            pl.BlockSpec((f, e), lambda i: (0, 0)),
            pl.BlockSpec((e, 1), lambda i: (0, 0)),
        ],
        out_specs=[
            pl.BlockSpec((e, _U_BLK), lambda i: (0, i)),
            pl.BlockSpec((e, _M_BLK), lambda i: (0, i)),
        ],
        out_shape=[
            jax.ShapeDtypeStruct((e, nu), jnp.float32),
            jax.ShapeDtypeStruct((e, n), jnp.float32),
        ],
    )(movie_x, u_t, W, b.reshape(-1, 1))
    return (user_t.T, movie_t.T)


# speedup vs baseline: 1.0028x; 1.0028x over previous
import jax
import jax.numpy as jnp
from jax.experimental import pallas as pl

_GRID = 20
_U_BLK = 51200
_M_BLK = 5120


def _fused_kernel(x_ref, u_ref, w_ref, b_ref, uo_ref, mo_ref):
    uo_ref[...] = u_ref[...]
    acc = jax.lax.dot_general(
        w_ref[...], x_ref[...],
        dimension_numbers=(((0,), (1,)), ((), ())),
        preferred_element_type=jnp.float32,
    )
    mo_ref[...] = jnp.maximum(acc + b_ref[...], 0.0)


def kernel(movie_x, user_emb_weight, W, b):
    n, f = movie_x.shape
    nu, e = user_emb_weight.shape
    u_t = user_emb_weight.T
    user_t, movie_t = pl.pallas_call(
        _fused_kernel,
        grid=(_GRID,),
        in_specs=[
            pl.BlockSpec((_M_BLK, f), lambda i: (i, 0)),
            pl.BlockSpec((e, _U_BLK), lambda i: (0, i)),
            pl.BlockSpec((f, e), lambda i: (0, 0)),
            pl.BlockSpec((e, 1), lambda i: (0, 0)),
        ],
        out_specs=[
            pl.BlockSpec((e, _U_BLK), lambda i: (0, i)),
            pl.BlockSpec((e, _M_BLK), lambda i: (0, i)),
        ],
        out_shape=[
            jax.ShapeDtypeStruct((e, nu), jnp.float32),
            jax.ShapeDtypeStruct((e, n), jnp.float32),
        ],
    )(movie_x, u_t, W, b.reshape(-1, 1))
    return (user_t.T, movie_t.T)
